# baseline (speedup 1.0000x reference)
import jax
import jax.numpy as jnp
from jax import lax
from jax.experimental import pallas as pl
from jax.experimental.pallas import tpu as pltpu
from jax.experimental.pallas import tpu_sc as plsc

_B, _C, _D = 1024, 128, 256
_R, _RC = 8, 16
_RSC = 2

_NC, _NS = 2, 16
_NW = _NC * _NS
_BPW = _B // _NW
_BCH = 4
_NCH = _BPW // _BCH
_NBUF = 5


def _sc_body(x_hbm, *refs):
    outs = refs[:_RSC]
    buf = refs[_RSC]
    in_sem = refs[_RSC + 1]
    out_sem = refs[_RSC + 2]
    wid = lax.axis_index("s") * _NC + lax.axis_index("c")
    base = wid * _BPW

    tiles = [(k, j) for k in range(_RSC) for j in range(_NCH)]
    n = len(tiles)

    def start_in(i):
        k, j = tiles[i]
        return pltpu.async_copy(
            x_hbm.at[pl.ds(base + j * _BCH, _BCH), pl.ds(k * _RC, _RC)],
            buf.at[i % _NBUF],
            in_sem,
        )

    def start_out(i):
        k, j = tiles[i]
        return pltpu.async_copy(
            buf.at[i % _NBUF],
            outs[k].at[pl.ds(base + j * _BCH, _BCH)],
            out_sem,
        )

    ahead = _NBUF - 1
    in_copies = {i: start_in(i) for i in range(min(ahead, n))}
    pending = {}
    for i in range(n):
        in_copies.pop(i).wait()
        pending[i] = start_out(i)
        if i + ahead - _NBUF in pending:
            pending.pop(i + ahead - _NBUF).wait()
        if i + ahead < n:
            in_copies[i + ahead] = start_in(i + ahead)
    for c in pending.values():
        c.wait()


_sc_call = pl.kernel(
    _sc_body,
    out_type=tuple(
        jax.ShapeDtypeStruct((_B, _RC, _D), jnp.float32) for _ in range(_RSC)
    ),
    mesh=plsc.VectorSubcoreMesh(core_axis_name="c", subcore_axis_name="s"),
    scratch_types=[
        pltpu.VMEM((_NBUF, _BCH, _RC, _D), jnp.float32),
        pltpu.SemaphoreType.DMA,
        pltpu.SemaphoreType.DMA,
    ],
)

_NTC = _R - _RSC
_TB = 32
_GRID = _B // _TB


def _tc_body(*refs):
    in_refs = refs[:_NTC]
    out_refs = refs[_NTC:]
    for k in range(_NTC):
        out_refs[k][...] = in_refs[k][...]


_tc_call = pl.pallas_call(
    _tc_body,
    grid=(_GRID,),
    in_specs=[
        pl.BlockSpec((_TB, _RC, _D), lambda i, k=k: (i, _RSC + k, 0))
        for k in range(_NTC)
    ],
    out_specs=[pl.BlockSpec((_TB, _RC, _D), lambda i: (i, 0, 0))] * _NTC,
    out_shape=tuple(
        jax.ShapeDtypeStruct((_B, _RC, _D), jnp.float32) for _ in range(_NTC)
    ),
)


@jax.jit
def kernel(x):
    sc_outs = _sc_call(x)
    tc_outs = _tc_call(*([x] * _NTC))
    return tuple(sc_outs) + tuple(tc_outs)


# speedup vs baseline: 1.0168x; 1.0168x over previous
import jax
import jax.numpy as jnp
from jax import lax
from jax.experimental import pallas as pl
from jax.experimental.pallas import tpu as pltpu
from jax.experimental.pallas import tpu_sc as plsc

_B, _C, _D = 1024, 128, 256
_R, _RC = 8, 16
_RSC = 1

_NC, _NS = 2, 16
_NW = _NC * _NS
_BPW = _B // _NW
_BCH = 4
_NCH = _BPW // _BCH
_NBUF = 5


def _sc_body(x_hbm, *refs):
    outs = refs[:_RSC]
    buf = refs[_RSC]
    in_sem = refs[_RSC + 1]
    out_sem = refs[_RSC + 2]
    wid = lax.axis_index("s") * _NC + lax.axis_index("c")
    base = wid * _BPW

    tiles = [(k, j) for k in range(_RSC) for j in range(_NCH)]
    n = len(tiles)

    def start_in(i):
        k, j = tiles[i]
        return pltpu.async_copy(
            x_hbm.at[pl.ds(base + j * _BCH, _BCH), pl.ds(k * _RC, _RC)],
            buf.at[i % _NBUF],
            in_sem,
        )

    def start_out(i):
        k, j = tiles[i]
        return pltpu.async_copy(
            buf.at[i % _NBUF],
            outs[k].at[pl.ds(base + j * _BCH, _BCH)],
            out_sem,
        )

    ahead = _NBUF - 1
    in_copies = {i: start_in(i) for i in range(min(ahead, n))}
    pending = {}
    for i in range(n):
        in_copies.pop(i).wait()
        pending[i] = start_out(i)
        if i + ahead - _NBUF in pending:
            pending.pop(i + ahead - _NBUF).wait()
        if i + ahead < n:
            in_copies[i + ahead] = start_in(i + ahead)
    for c in pending.values():
        c.wait()


_sc_call = pl.kernel(
    _sc_body,
    out_type=tuple(
        jax.ShapeDtypeStruct((_B, _RC, _D), jnp.float32) for _ in range(_RSC)
    ),
    mesh=plsc.VectorSubcoreMesh(core_axis_name="c", subcore_axis_name="s"),
    scratch_types=[
        pltpu.VMEM((_NBUF, _BCH, _RC, _D), jnp.float32),
        pltpu.SemaphoreType.DMA,
        pltpu.SemaphoreType.DMA,
    ],
)

_NTC = _R - _RSC
_TB = 32
_GRID = _B // _TB


def _tc_body(*refs):
    in_refs = refs[:_NTC]
    out_refs = refs[_NTC:]
    for k in range(_NTC):
        out_refs[k][...] = in_refs[k][...]


_tc_call = pl.pallas_call(
    _tc_body,
    grid=(_GRID,),
    in_specs=[
        pl.BlockSpec((_TB, _RC, _D), lambda i, k=k: (i, _RSC + k, 0))
        for k in range(_NTC)
    ],
    out_specs=[pl.BlockSpec((_TB, _RC, _D), lambda i: (i, 0, 0))] * _NTC,
    out_shape=tuple(
        jax.ShapeDtypeStruct((_B, _RC, _D), jnp.float32) for _ in range(_NTC)
    ),
)


@jax.jit
def kernel(x):
    sc_outs = _sc_call(x)
    tc_outs = _tc_call(*([x] * _NTC))
    return tuple(sc_outs) + tuple(tc_outs)
